# baseline (speedup 1.0000x reference)
import functools

import jax
import jax.numpy as jnp
from jax import lax
from jax.experimental import pallas as pl
from jax.experimental.pallas import tpu as pltpu
from jax.experimental.pallas import tpu_sc as plsc

NU = 25000
PAD = 25600
TBL = 2 * PAD
D = 64
E = 800000
BATCH = 1024
NC, NS, L = 2, 16, 16
EPT = E // NS
CH = 80
STRIPE = PAD // NS
BLK = 2000
CPB = BLK // CH
NB = EPT // BLK

_mesh = plsc.VectorSubcoreMesh(core_axis_name="c", subcore_axis_name="s")
_sc_params = pltpu.CompilerParams(needs_layout_passes=False,
                                  use_tc_tiling_on_sc=False)


def _prop_body(x_hbm, src_hbm, dst_hbm, w_hbm, out_hbm,
               src_b, dst_b, w_b,
               src_v0, src_v1, idx_v0, idx_v1, rows0, rows1,
               sem0, sem1, acc):
    c = lax.axis_index("c")
    s = lax.axis_index("s")

    def zrow(r, _):
        z = jnp.zeros((L,), jnp.float32)
        for q in range(D // L):
            rows0[r, pl.ds(q * L, L)] = z
        return 0
    lax.fori_loop(0, CH, zrow, 0)

    def zcp(k, _):
        pltpu.sync_copy(rows0, acc.at[pl.ds(s * STRIPE + k * CH, CH)])
        return 0
    lax.fori_loop(0, STRIPE // CH, zcp, 0)
    plsc.subcore_barrier()

    nu = jnp.int32(NU)
    base_node = c * nu

    def remap(ck, src_v, idx_v):
        def body(i, _):
            sl_b = pl.ds(ck * CH + i * L, L)
            sl = pl.ds(i * L, L)
            sv = src_b[sl_b]
            src_v[sl] = jnp.where(sv >= nu, sv + (PAD - NU), sv)
            dv = dst_b[sl_b]
            ld = dv - base_node
            ok = (ld >= 0) & (ld < nu)
            idx_v[sl] = jnp.where(ok, ld, nu + (dv & 511))
            return 0
        lax.fori_loop(0, CH // L, body, 0)

    def scale_scatter(ck, rows, idx_v):
        wbase = ck * CH

        def body(g, _):
            w16 = w_b[pl.ds(wbase + g * L, L)]
            for j in range(L):
                wspl = jnp.take_along_axis(
                    w16, jnp.full((L,), j, jnp.int32),
                    axis=0, mode="promise_in_bounds")
                r = g * L + j
                for q in range(D // L):
                    sl = pl.ds(q * L, L)
                    rows[r, sl] = rows[r, sl] * wspl
            return 0
        lax.fori_loop(0, CH // L, body, 0)
        pltpu.sync_copy(rows, acc.at[idx_v], add=True)

    def block(b, _):
        base = s * EPT + b * BLK
        pltpu.sync_copy(src_hbm.at[pl.ds(base, BLK)], src_b)
        pltpu.sync_copy(dst_hbm.at[pl.ds(base, BLK)], dst_b)
        pltpu.sync_copy(w_hbm.at[pl.ds(base, BLK)], w_b)

        remap(0, src_v0, idx_v0)
        pltpu.async_copy(x_hbm.at[src_v0], rows0, sem0)

        def pair(j, _):
            remap(2 * j + 1, src_v1, idx_v1)
            pltpu.async_copy(x_hbm.at[src_v1], rows1, sem1)
            pltpu.make_async_copy(x_hbm.at[src_v0], rows0, sem0).wait()
            scale_scatter(2 * j, rows0, idx_v0)
            remap(2 * j + 2, src_v0, idx_v0)
            pltpu.async_copy(x_hbm.at[src_v0], rows0, sem0)
            pltpu.make_async_copy(x_hbm.at[src_v1], rows1, sem1).wait()
            scale_scatter(2 * j + 1, rows1, idx_v1)
            return 0
        lax.fori_loop(0, (CPB - 1) // 2, pair, 0)

        pltpu.make_async_copy(x_hbm.at[src_v0], rows0, sem0).wait()
        scale_scatter(CPB - 1, rows0, idx_v0)
        return 0

    lax.fori_loop(0, NB, block, 0)
    plsc.subcore_barrier()

    pltpu.sync_copy(acc.at[pl.ds(s * STRIPE, STRIPE)],
                    out_hbm.at[pl.ds(c * PAD + s * STRIPE, STRIPE)])


_propagate = pl.kernel(
    _prop_body,
    out_type=jax.ShapeDtypeStruct((TBL, D), jnp.float32),
    mesh=_mesh,
    compiler_params=_sc_params,
    scratch_types=[
        pltpu.VMEM((BLK,), jnp.int32),
        pltpu.VMEM((BLK,), jnp.int32),
        pltpu.VMEM((BLK,), jnp.float32),
        pltpu.VMEM((CH,), jnp.int32),
        pltpu.VMEM((CH,), jnp.int32),
        pltpu.VMEM((CH,), jnp.int32),
        pltpu.VMEM((CH,), jnp.int32),
        pltpu.VMEM((CH, D), jnp.float32),
        pltpu.VMEM((CH, D), jnp.float32),
        pltpu.SemaphoreType.DMA,
        pltpu.SemaphoreType.DMA,
        pltpu.VMEM_SHARED((PAD, D), jnp.float32),
    ],
)

UPW = BATCH // (NC * NS)


def _gusers_body(u_hbm, x0, x1, x2, x3, out_hbm, uidx_v, a_v, b_v):
    c = lax.axis_index("c")
    s = lax.axis_index("s")
    w = s * NC + c
    base = w * UPW
    pltpu.sync_copy(u_hbm.at[pl.ds(base, UPW)], uidx_v)
    pltpu.sync_copy(x0.at[uidx_v], a_v)
    for t in (x1, x2, x3):
        pltpu.sync_copy(t.at[uidx_v], b_v)

        def addr(r, _):
            for q in range(D // L):
                sl = pl.ds(q * L, L)
                a_v[r, sl] = a_v[r, sl] + b_v[r, sl]
            return 0
        lax.fori_loop(0, UPW, addr, 0)
    pltpu.sync_copy(a_v, out_hbm.at[pl.ds(base, UPW)])


_gather_users = pl.kernel(
    _gusers_body,
    out_type=jax.ShapeDtypeStruct((BATCH, D), jnp.float32),
    mesh=_mesh,
    compiler_params=_sc_params,
    scratch_types=[
        pltpu.VMEM((UPW,), jnp.int32),
        pltpu.VMEM((UPW, D), jnp.float32),
        pltpu.VMEM((UPW, D), jnp.float32),
    ],
)

SUMB = 1024


def _sum_body(a, b, c, d, o):
    o[...] = a[...] + b[...] + c[...] + d[...]


def _sum_items(x0, x1, x2, x3):
    in_spec = pl.BlockSpec((SUMB, D), lambda j: (PAD // SUMB + j, 0))
    return pl.pallas_call(
        _sum_body,
        grid=(PAD // SUMB,),
        in_specs=[in_spec, in_spec, in_spec, in_spec],
        out_specs=pl.BlockSpec((SUMB, D), lambda j: (j, 0)),
        out_shape=jax.ShapeDtypeStruct((PAD, D), jnp.float32),
    )(x0, x1, x2, x3)


MB = 64


def _mm_body(u_ref, it_ref, o_ref):
    acc = lax.dot_general(u_ref[...], it_ref[:NU, :],
                          (((1,), (1,)), ((), ())),
                          preferred_element_type=jnp.float32)
    o_ref[...] = jax.nn.sigmoid(acc * 0.0625)


def _rating(u_s, items_sum):
    return pl.pallas_call(
        _mm_body,
        grid=(BATCH // MB,),
        in_specs=[pl.BlockSpec((MB, D), lambda j: (j, 0)),
                  pl.BlockSpec((PAD, D), lambda j: (0, 0))],
        out_specs=pl.BlockSpec((MB, NU), lambda j: (j, 0)),
        out_shape=jax.ShapeDtypeStruct((BATCH, NU), jnp.float32),
    )(u_s, items_sum)


@jax.jit
def kernel(users, edge_index, edge_weight, user_emb, item_emb):
    users = users.astype(jnp.int32)
    src = edge_index[0].astype(jnp.int32)
    dst = edge_index[1].astype(jnp.int32)
    w = edge_weight.astype(jnp.float32)

    x0 = jnp.zeros((TBL, D), jnp.float32)
    x0 = x0.at[:NU].set(user_emb).at[PAD:PAD + NU].set(item_emb)

    x1 = _propagate(x0, src, dst, w)
    x2 = _propagate(x1, src, dst, w)
    x3 = _propagate(x2, src, dst, w)

    i_sum = _sum_items(x0, x1, x2, x3)
    u_s = _gather_users(users, x0, x1, x2, x3)
    return _rating(u_s, i_sum)


# speedup vs baseline: 2.0360x; 2.0360x over previous
import functools

import jax
import jax.numpy as jnp
from jax import lax
from jax.experimental import pallas as pl
from jax.experimental.pallas import tpu as pltpu
from jax.experimental.pallas import tpu_sc as plsc

NU = 25000
PAD = 25600
TBL = 2 * PAD
D = 64
E = 800000
BATCH = 1024
NC, NS, L = 2, 16, 16
EPT = E // NS
CH = 80
STRIPE = PAD // NS
BLK = 2000
CPB = BLK // CH
NB = EPT // BLK

_mesh = plsc.VectorSubcoreMesh(core_axis_name="c", subcore_axis_name="s")
_sc_params = pltpu.CompilerParams(needs_layout_passes=False,
                                  use_tc_tiling_on_sc=False)


def _prop_body(x_hbm, src_hbm, dst_hbm, w_hbm, out_hbm,
               src_b, dst_b, w_b,
               sv0, sv1, sv2, sv3, iv0, iv1, iv2, iv3,
               r0, r1, r2, r3,
               g0, g1, g2, g3, s0, s1, s2, s3, acc):
    SV = (sv0, sv1, sv2, sv3)
    IV = (iv0, iv1, iv2, iv3)
    RW = (r0, r1, r2, r3)
    GS = (g0, g1, g2, g3)
    SS = (s0, s1, s2, s3)
    c = lax.axis_index("c")
    s = lax.axis_index("s")
    nu = jnp.int32(NU)

    def zrow(r, _):
        z = jnp.zeros((L,), jnp.float32)
        for q in range(D // L):
            r0[r, pl.ds(q * L, L)] = z
        return 0
    lax.fori_loop(0, CH, zrow, 0)

    def zcp(k, _):
        pltpu.sync_copy(r0, acc.at[pl.ds(s * STRIPE + k * CH, CH)])
        return 0
    lax.fori_loop(0, STRIPE // CH, zcp, 0)

    trash16 = nu + lax.iota(jnp.int32, L)
    for ivb in IV:
        def tini(i, _, ivb=ivb):
            ivb[pl.ds(i * L, L)] = trash16
            return 0
        lax.fori_loop(0, CH // L, tini, 0)
    plsc.subcore_barrier()
    for b in range(4):
        pltpu.async_copy(RW[b], acc.at[IV[b]], SS[b], add=True)

    base_node = c * nu

    def remap(ck, src_v, idx_v):
        def body(i, _):
            sl_b = pl.ds(ck * CH + i * L, L)
            sl = pl.ds(i * L, L)
            sv = src_b[sl_b]
            src_v[sl] = jnp.where(sv >= nu, sv + (PAD - NU), sv)
            dv = dst_b[sl_b]
            ld = dv - base_node
            ok = (ld >= 0) & (ld < nu)
            idx_v[sl] = jnp.where(ok, ld, nu + (dv & 511))
            return 0
        lax.fori_loop(0, CH // L, body, 0)

    def scale(ck, rows):
        wbase = ck * CH

        def body(jj, _):
            for u in range(4):
                j = jj * 4 + u
                wspl = plsc.load_gather(
                    w_b, [jnp.zeros((L,), jnp.int32) + (wbase + j)])
                for q in range(D // L):
                    sl = pl.ds(q * L, L)
                    rows[j, sl] = rows[j, sl] * wspl
            return 0
        lax.fori_loop(0, CH // 4, body, 0)

    def wait_scatter(b):
        pltpu.make_async_copy(RW[b], acc.at[IV[b]], SS[b]).wait()

    def wait_gather(b):
        pltpu.make_async_copy(x_hbm.at[SV[b]], RW[b], GS[b]).wait()

    def slot(ck, t):
        b2 = (t + 2) % 4

        @pl.when(ck + 2 < CPB)
        def _():
            wait_scatter(b2)
            remap(ck + 2, SV[b2], IV[b2])
            pltpu.async_copy(x_hbm.at[SV[b2]], RW[b2], GS[b2])
        wait_gather(t)
        scale(ck, RW[t])
        pltpu.async_copy(RW[t], acc.at[IV[t]], SS[t], add=True)

    def block(blk, _):
        base = s * EPT + blk * BLK
        pltpu.sync_copy(src_hbm.at[pl.ds(base, BLK)], src_b)
        pltpu.sync_copy(dst_hbm.at[pl.ds(base, BLK)], dst_b)
        pltpu.sync_copy(w_hbm.at[pl.ds(base, BLK)], w_b)

        for b in (0, 1):
            wait_scatter(b)
            remap(b, SV[b], IV[b])
            pltpu.async_copy(x_hbm.at[SV[b]], RW[b], GS[b])

        def quad(q, _):
            for t in range(4):
                slot(4 * q + t, t)
            return 0
        lax.fori_loop(0, CPB // 4, quad, 0)

        wait_gather(0)
        scale(CPB - 1, RW[0])
        pltpu.async_copy(RW[0], acc.at[IV[0]], SS[0], add=True)
        return 0

    lax.fori_loop(0, NB, block, 0)

    for b in range(4):
        wait_scatter(b)
    plsc.subcore_barrier()

    pltpu.sync_copy(acc.at[pl.ds(s * STRIPE, STRIPE)],
                    out_hbm.at[pl.ds(c * PAD + s * STRIPE, STRIPE)])


_propagate = pl.kernel(
    _prop_body,
    out_type=jax.ShapeDtypeStruct((TBL, D), jnp.float32),
    mesh=_mesh,
    compiler_params=_sc_params,
    scratch_types=[
        pltpu.VMEM((BLK,), jnp.int32),
        pltpu.VMEM((BLK,), jnp.int32),
        pltpu.VMEM((BLK,), jnp.float32),
        pltpu.VMEM((CH,), jnp.int32),
        pltpu.VMEM((CH,), jnp.int32),
        pltpu.VMEM((CH,), jnp.int32),
        pltpu.VMEM((CH,), jnp.int32),
        pltpu.VMEM((CH,), jnp.int32),
        pltpu.VMEM((CH,), jnp.int32),
        pltpu.VMEM((CH,), jnp.int32),
        pltpu.VMEM((CH,), jnp.int32),
        pltpu.VMEM((CH, D), jnp.float32),
        pltpu.VMEM((CH, D), jnp.float32),
        pltpu.VMEM((CH, D), jnp.float32),
        pltpu.VMEM((CH, D), jnp.float32),
        pltpu.SemaphoreType.DMA,
        pltpu.SemaphoreType.DMA,
        pltpu.SemaphoreType.DMA,
        pltpu.SemaphoreType.DMA,
        pltpu.SemaphoreType.DMA,
        pltpu.SemaphoreType.DMA,
        pltpu.SemaphoreType.DMA,
        pltpu.SemaphoreType.DMA,
        pltpu.VMEM_SHARED((PAD, D), jnp.float32),
    ],
)

UPW = BATCH // (NC * NS)


def _gusers_body(u_hbm, x0, x1, x2, x3, out_hbm, uidx_v, a_v, b_v):
    c = lax.axis_index("c")
    s = lax.axis_index("s")
    w = s * NC + c
    base = w * UPW
    pltpu.sync_copy(u_hbm.at[pl.ds(base, UPW)], uidx_v)
    pltpu.sync_copy(x0.at[uidx_v], a_v)
    for t in (x1, x2, x3):
        pltpu.sync_copy(t.at[uidx_v], b_v)

        def addr(r, _):
            for q in range(D // L):
                sl = pl.ds(q * L, L)
                a_v[r, sl] = a_v[r, sl] + b_v[r, sl]
            return 0
        lax.fori_loop(0, UPW, addr, 0)
    pltpu.sync_copy(a_v, out_hbm.at[pl.ds(base, UPW)])


_gather_users = pl.kernel(
    _gusers_body,
    out_type=jax.ShapeDtypeStruct((BATCH, D), jnp.float32),
    mesh=_mesh,
    compiler_params=_sc_params,
    scratch_types=[
        pltpu.VMEM((UPW,), jnp.int32),
        pltpu.VMEM((UPW, D), jnp.float32),
        pltpu.VMEM((UPW, D), jnp.float32),
    ],
)

SUMB = 1024


def _sum_body(a, b, c, d, o):
    o[...] = a[...] + b[...] + c[...] + d[...]


def _sum_items(x0, x1, x2, x3):
    in_spec = pl.BlockSpec((SUMB, D), lambda j: (PAD // SUMB + j, 0))
    return pl.pallas_call(
        _sum_body,
        grid=(PAD // SUMB,),
        in_specs=[in_spec, in_spec, in_spec, in_spec],
        out_specs=pl.BlockSpec((SUMB, D), lambda j: (j, 0)),
        out_shape=jax.ShapeDtypeStruct((PAD, D), jnp.float32),
    )(x0, x1, x2, x3)


MB = 64


def _mm_body(u_ref, it_ref, o_ref):
    acc = lax.dot_general(u_ref[...], it_ref[:NU, :],
                          (((1,), (1,)), ((), ())),
                          preferred_element_type=jnp.float32)
    o_ref[...] = jax.nn.sigmoid(acc * 0.0625)


def _rating(u_s, items_sum):
    return pl.pallas_call(
        _mm_body,
        grid=(BATCH // MB,),
        in_specs=[pl.BlockSpec((MB, D), lambda j: (j, 0)),
                  pl.BlockSpec((PAD, D), lambda j: (0, 0))],
        out_specs=pl.BlockSpec((MB, NU), lambda j: (j, 0)),
        out_shape=jax.ShapeDtypeStruct((BATCH, NU), jnp.float32),
    )(u_s, items_sum)


@jax.jit
def kernel(users, edge_index, edge_weight, user_emb, item_emb):
    users = users.astype(jnp.int32)
    src = edge_index[0].astype(jnp.int32)
    dst = edge_index[1].astype(jnp.int32)
    w = edge_weight.astype(jnp.float32)

    x0 = jnp.zeros((TBL, D), jnp.float32)
    x0 = x0.at[:NU].set(user_emb).at[PAD:PAD + NU].set(item_emb)

    x1 = _propagate(x0, src, dst, w)
    x2 = _propagate(x1, src, dst, w)
    x3 = _propagate(x2, src, dst, w)

    i_sum = _sum_items(x0, x1, x2, x3)
    u_s = _gather_users(users, x0, x1, x2, x3)
    return _rating(u_s, i_sum)


# speedup vs baseline: 2.1549x; 1.0584x over previous
import functools

import jax
import jax.numpy as jnp
from jax import lax
from jax.experimental import pallas as pl
from jax.experimental.pallas import tpu as pltpu
from jax.experimental.pallas import tpu_sc as plsc

NU = 25000
PAD = 25600
TBL = 2 * PAD
D = 64
E = 800000
BATCH = 1024
NC, NS, L = 2, 16, 16
EPT = E // NS
CH = 80
STRIPE = PAD // NS
BLK = 2000
CPB = BLK // CH
NB = EPT // BLK

_mesh = plsc.VectorSubcoreMesh(core_axis_name="c", subcore_axis_name="s")
_sc_params = pltpu.CompilerParams(needs_layout_passes=False,
                                  use_tc_tiling_on_sc=False)


def _prop_body(x_hbm, src_hbm, dst_hbm, w_hbm, out_hbm,
               src_b, dst_b, w_b,
               sv0, sv1, sv2, sv3, iv0, iv1, iv2, iv3,
               wv0, wv1, wv2, wv3,
               r0, r1, r2, r3,
               g0, g1, g2, g3, s0, s1, s2, s3, esem, acc):
    SV = (sv0, sv1, sv2, sv3)
    IV = (iv0, iv1, iv2, iv3)
    WV = (wv0, wv1, wv2, wv3)
    RW = (r0, r1, r2, r3)
    GS = (g0, g1, g2, g3)
    SS = (s0, s1, s2, s3)
    c = lax.axis_index("c")
    s = lax.axis_index("s")
    nu = jnp.int32(NU)

    def zrow(r, _):
        z = jnp.zeros((L,), jnp.float32)
        for q in range(D // L):
            r0[r, pl.ds(q * L, L)] = z
        return 0
    lax.fori_loop(0, CH, zrow, 0)

    def zcp(k, _):
        pltpu.sync_copy(r0, acc.at[pl.ds(s * STRIPE + k * CH, CH)])
        return 0
    lax.fori_loop(0, STRIPE // CH, zcp, 0)

    trash16 = nu + lax.iota(jnp.int32, L)
    for ivb in IV:
        def tini(i, _, ivb=ivb):
            ivb[pl.ds(i * L, L)] = trash16
            return 0
        lax.fori_loop(0, CH // L, tini, 0)
    plsc.subcore_barrier()
    for b in range(4):
        pltpu.async_copy(RW[b], acc.at[IV[b]], SS[b], add=True)

    base_node = c * nu

    def remap(ck, src_v, idx_v, w_v):
        def body(i, _):
            sl_b = pl.ds(ck * CH + i * L, L)
            sl = pl.ds(i * L, L)
            sv = src_b[sl_b]
            src_v[sl] = jnp.where(sv >= nu, sv + (PAD - NU), sv)
            dv = dst_b[sl_b]
            ld = dv - base_node
            ok = (ld >= 0) & (ld < nu)
            idx_v[sl] = jnp.where(ok, ld, nu + (dv & 511))
            w_v[sl] = w_b[sl_b]
            return 0
        lax.fori_loop(0, CH // L, body, 0)

    def scale(rows, w_v):
        def body(jj, _):
            for u in range(4):
                j = jj * 4 + u
                wspl = plsc.load_gather(
                    w_v, [jnp.zeros((L,), jnp.int32) + j])
                for q in range(D // L):
                    sl = pl.ds(q * L, L)
                    rows[j, sl] = rows[j, sl] * wspl
            return 0
        lax.fori_loop(0, CH // 4, body, 0)

    def wait_scatter(b):
        pltpu.make_async_copy(RW[b], acc.at[IV[b]], SS[b]).wait()

    def wait_gather(b):
        pltpu.make_async_copy(x_hbm.at[SV[b]], RW[b], GS[b]).wait()

    def slot(ck, t):
        b2 = (t + 2) % 4

        @pl.when(ck + 2 < CPB)
        def _():
            wait_scatter(b2)
            remap(ck + 2, SV[b2], IV[b2], WV[b2])
            pltpu.async_copy(x_hbm.at[SV[b2]], RW[b2], GS[b2])
        wait_gather(t)
        scale(RW[t], WV[t])
        pltpu.async_copy(RW[t], acc.at[IV[t]], SS[t], add=True)

    def edge_load(blk, copy):
        base = s * EPT + blk * BLK
        copy(src_hbm.at[pl.ds(base, BLK)], src_b)
        copy(dst_hbm.at[pl.ds(base, BLK)], dst_b)
        copy(w_hbm.at[pl.ds(base, BLK)], w_b)

    edge_load(0, lambda a, b: pltpu.async_copy(a, b, esem))

    def block(blk, _):
        edge_load(blk, lambda a, b: pltpu.make_async_copy(a, b, esem).wait())

        for b in (0, 1):
            wait_scatter(b)
            remap(b, SV[b], IV[b], WV[b])
            pltpu.async_copy(x_hbm.at[SV[b]], RW[b], GS[b])

        def quad(q, _):
            for t in range(4):
                slot(4 * q + t, t)
            return 0
        lax.fori_loop(0, CPB // 4, quad, 0)

        @pl.when(blk + 1 < NB)
        def _():
            edge_load(blk + 1, lambda a, b: pltpu.async_copy(a, b, esem))

        wait_gather(0)
        scale(RW[0], WV[0])
        pltpu.async_copy(RW[0], acc.at[IV[0]], SS[0], add=True)
        return 0

    lax.fori_loop(0, NB, block, 0)

    for b in range(4):
        wait_scatter(b)
    plsc.subcore_barrier()

    pltpu.sync_copy(acc.at[pl.ds(s * STRIPE, STRIPE)],
                    out_hbm.at[pl.ds(c * PAD + s * STRIPE, STRIPE)])


_propagate = pl.kernel(
    _prop_body,
    out_type=jax.ShapeDtypeStruct((TBL, D), jnp.float32),
    mesh=_mesh,
    compiler_params=_sc_params,
    scratch_types=[
        pltpu.VMEM((BLK,), jnp.int32),
        pltpu.VMEM((BLK,), jnp.int32),
        pltpu.VMEM((BLK,), jnp.float32),
        pltpu.VMEM((CH,), jnp.int32),
        pltpu.VMEM((CH,), jnp.int32),
        pltpu.VMEM((CH,), jnp.int32),
        pltpu.VMEM((CH,), jnp.int32),
        pltpu.VMEM((CH,), jnp.int32),
        pltpu.VMEM((CH,), jnp.int32),
        pltpu.VMEM((CH,), jnp.int32),
        pltpu.VMEM((CH,), jnp.int32),
        pltpu.VMEM((CH,), jnp.float32),
        pltpu.VMEM((CH,), jnp.float32),
        pltpu.VMEM((CH,), jnp.float32),
        pltpu.VMEM((CH,), jnp.float32),
        pltpu.VMEM((CH, D), jnp.float32),
        pltpu.VMEM((CH, D), jnp.float32),
        pltpu.VMEM((CH, D), jnp.float32),
        pltpu.VMEM((CH, D), jnp.float32),
        pltpu.SemaphoreType.DMA,
        pltpu.SemaphoreType.DMA,
        pltpu.SemaphoreType.DMA,
        pltpu.SemaphoreType.DMA,
        pltpu.SemaphoreType.DMA,
        pltpu.SemaphoreType.DMA,
        pltpu.SemaphoreType.DMA,
        pltpu.SemaphoreType.DMA,
        pltpu.SemaphoreType.DMA,
        pltpu.VMEM_SHARED((PAD, D), jnp.float32),
    ],
)

UPW = BATCH // (NC * NS)


def _gusers_body(u_hbm, x0, x1, x2, x3, out_hbm, uidx_v, a_v, b_v):
    c = lax.axis_index("c")
    s = lax.axis_index("s")
    w = s * NC + c
    base = w * UPW
    pltpu.sync_copy(u_hbm.at[pl.ds(base, UPW)], uidx_v)
    pltpu.sync_copy(x0.at[uidx_v], a_v)
    for t in (x1, x2, x3):
        pltpu.sync_copy(t.at[uidx_v], b_v)

        def addr(r, _):
            for q in range(D // L):
                sl = pl.ds(q * L, L)
                a_v[r, sl] = a_v[r, sl] + b_v[r, sl]
            return 0
        lax.fori_loop(0, UPW, addr, 0)
    pltpu.sync_copy(a_v, out_hbm.at[pl.ds(base, UPW)])


_gather_users = pl.kernel(
    _gusers_body,
    out_type=jax.ShapeDtypeStruct((BATCH, D), jnp.float32),
    mesh=_mesh,
    compiler_params=_sc_params,
    scratch_types=[
        pltpu.VMEM((UPW,), jnp.int32),
        pltpu.VMEM((UPW, D), jnp.float32),
        pltpu.VMEM((UPW, D), jnp.float32),
    ],
)

SUMB = 1024


def _sum_body(a, b, c, d, o):
    o[...] = a[...] + b[...] + c[...] + d[...]


def _sum_items(x0, x1, x2, x3):
    in_spec = pl.BlockSpec((SUMB, D), lambda j: (PAD // SUMB + j, 0))
    return pl.pallas_call(
        _sum_body,
        grid=(PAD // SUMB,),
        in_specs=[in_spec, in_spec, in_spec, in_spec],
        out_specs=pl.BlockSpec((SUMB, D), lambda j: (j, 0)),
        out_shape=jax.ShapeDtypeStruct((PAD, D), jnp.float32),
    )(x0, x1, x2, x3)


MB = 64


def _mm_body(u_ref, it_ref, o_ref):
    acc = lax.dot_general(u_ref[...], it_ref[:NU, :],
                          (((1,), (1,)), ((), ())),
                          preferred_element_type=jnp.float32)
    o_ref[...] = jax.nn.sigmoid(acc * 0.0625)


def _rating(u_s, items_sum):
    return pl.pallas_call(
        _mm_body,
        grid=(BATCH // MB,),
        in_specs=[pl.BlockSpec((MB, D), lambda j: (j, 0)),
                  pl.BlockSpec((PAD, D), lambda j: (0, 0))],
        out_specs=pl.BlockSpec((MB, NU), lambda j: (j, 0)),
        out_shape=jax.ShapeDtypeStruct((BATCH, NU), jnp.float32),
    )(u_s, items_sum)


@jax.jit
def kernel(users, edge_index, edge_weight, user_emb, item_emb):
    users = users.astype(jnp.int32)
    src = edge_index[0].astype(jnp.int32)
    dst = edge_index[1].astype(jnp.int32)
    w = edge_weight.astype(jnp.float32)

    x0 = jnp.zeros((TBL, D), jnp.float32)
    x0 = x0.at[:NU].set(user_emb).at[PAD:PAD + NU].set(item_emb)

    x1 = _propagate(x0, src, dst, w)
    x2 = _propagate(x1, src, dst, w)
    x3 = _propagate(x2, src, dst, w)

    i_sum = _sum_items(x0, x1, x2, x3)
    u_s = _gather_users(users, x0, x1, x2, x3)
    return _rating(u_s, i_sum)


# speedup vs baseline: 2.8739x; 1.3336x over previous
import functools

import jax
import jax.numpy as jnp
from jax import lax
from jax.experimental import pallas as pl
from jax.experimental.pallas import tpu as pltpu
from jax.experimental.pallas import tpu_sc as plsc

NU = 25000
PAD = 25600
TBL = 2 * PAD
D = 64
E = 800000
BATCH = 1024
NC, NS, L = 2, 16, 16
EPT = E // NS
CH = 80
STRIPE = PAD // NS
BLK = 2000
CPB = BLK // CH
NB = EPT // BLK

_mesh = plsc.VectorSubcoreMesh(core_axis_name="c", subcore_axis_name="s")
_sc_params = pltpu.CompilerParams(needs_layout_passes=False,
                                  use_tc_tiling_on_sc=False)


def _prop_body(x_hbm, src_hbm, dst_hbm, w_hbm, out_hbm,
               src_b, dst_b, w_b,
               sv0, sv1, sv2, sv3, iv0, iv1, iv2, iv3,
               wv0, wv1, wv2, wv3,
               r0, r1, r2, r3,
               g0, g1, g2, g3, s0, s1, s2, s3, esem, acc):
    SV = (sv0, sv1, sv2, sv3)
    IV = (iv0, iv1, iv2, iv3)
    WV = (wv0, wv1, wv2, wv3)
    RW = (r0, r1, r2, r3)
    GS = (g0, g1, g2, g3)
    SS = (s0, s1, s2, s3)
    c = lax.axis_index("c")
    s = lax.axis_index("s")
    nu = jnp.int32(NU)

    def zrow(r, _):
        z = jnp.zeros((L,), jnp.float32)
        for q in range(D // L):
            r0[r, pl.ds(q * L, L)] = z
        return 0
    lax.fori_loop(0, CH, zrow, 0)

    def zcp(k, _):
        pltpu.sync_copy(r0, acc.at[pl.ds(s * STRIPE + k * CH, CH)])
        return 0
    lax.fori_loop(0, STRIPE // CH, zcp, 0)

    trash16 = nu + lax.iota(jnp.int32, L)
    for ivb in IV:
        def tini(i, _, ivb=ivb):
            ivb[pl.ds(i * L, L)] = trash16
            return 0
        lax.fori_loop(0, CH // L, tini, 0)
    plsc.subcore_barrier()
    for b in range(4):
        pltpu.async_copy(RW[b], acc.at[IV[b]], SS[b], add=True)

    base_node = c * nu

    def remap(ck, src_v, idx_v, w_v):
        @plsc.parallel_loop(0, CH // L)
        def body(i):
            sl_b = pl.ds(ck * CH + i * L, L)
            sl = pl.ds(i * L, L)
            sv = src_b[sl_b]
            src_v[sl] = jnp.where(sv >= nu, sv + (PAD - NU), sv)
            dv = dst_b[sl_b]
            ld = dv - base_node
            ok = (ld >= 0) & (ld < nu)
            idx_v[sl] = jnp.where(ok, ld, nu + (dv & 511))
            w_v[sl] = w_b[sl_b]

    def scale(rows, w_v):
        @plsc.parallel_loop(0, CH // 4, unroll=2)
        def body(jj):
            for u in range(4):
                j = jj * 4 + u
                wspl = plsc.load_gather(
                    w_v, [jnp.zeros((L,), jnp.int32) + j])
                for q in range(D // L):
                    sl = pl.ds(q * L, L)
                    rows[j, sl] = rows[j, sl] * wspl

    def wait_scatter(b):
        pltpu.make_async_copy(RW[b], acc.at[IV[b]], SS[b]).wait()

    def wait_gather(b):
        pltpu.make_async_copy(x_hbm.at[SV[b]], RW[b], GS[b]).wait()

    def slot(ck, t):
        b2 = (t + 2) % 4

        @pl.when(ck + 2 < CPB)
        def _():
            wait_scatter(b2)
            remap(ck + 2, SV[b2], IV[b2], WV[b2])
            pltpu.async_copy(x_hbm.at[SV[b2]], RW[b2], GS[b2])
        wait_gather(t)
        scale(RW[t], WV[t])
        pltpu.async_copy(RW[t], acc.at[IV[t]], SS[t], add=True)

    def edge_load(blk, copy):
        base = s * EPT + blk * BLK
        copy(src_hbm.at[pl.ds(base, BLK)], src_b)
        copy(dst_hbm.at[pl.ds(base, BLK)], dst_b)
        copy(w_hbm.at[pl.ds(base, BLK)], w_b)

    edge_load(0, lambda a, b: pltpu.async_copy(a, b, esem))

    def block(blk, _):
        edge_load(blk, lambda a, b: pltpu.make_async_copy(a, b, esem).wait())

        for b in (0, 1):
            wait_scatter(b)
            remap(b, SV[b], IV[b], WV[b])
            pltpu.async_copy(x_hbm.at[SV[b]], RW[b], GS[b])

        def quad(q, _):
            for t in range(4):
                slot(4 * q + t, t)
            return 0
        lax.fori_loop(0, CPB // 4, quad, 0)

        @pl.when(blk + 1 < NB)
        def _():
            edge_load(blk + 1, lambda a, b: pltpu.async_copy(a, b, esem))

        wait_gather(0)
        scale(RW[0], WV[0])
        pltpu.async_copy(RW[0], acc.at[IV[0]], SS[0], add=True)
        return 0

    lax.fori_loop(0, NB, block, 0)

    for b in range(4):
        wait_scatter(b)
    plsc.subcore_barrier()

    pltpu.sync_copy(acc.at[pl.ds(s * STRIPE, STRIPE)],
                    out_hbm.at[pl.ds(c * PAD + s * STRIPE, STRIPE)])


_propagate = pl.kernel(
    _prop_body,
    out_type=jax.ShapeDtypeStruct((TBL, D), jnp.float32),
    mesh=_mesh,
    compiler_params=_sc_params,
    scratch_types=[
        pltpu.VMEM((BLK,), jnp.int32),
        pltpu.VMEM((BLK,), jnp.int32),
        pltpu.VMEM((BLK,), jnp.float32),
        pltpu.VMEM((CH,), jnp.int32),
        pltpu.VMEM((CH,), jnp.int32),
        pltpu.VMEM((CH,), jnp.int32),
        pltpu.VMEM((CH,), jnp.int32),
        pltpu.VMEM((CH,), jnp.int32),
        pltpu.VMEM((CH,), jnp.int32),
        pltpu.VMEM((CH,), jnp.int32),
        pltpu.VMEM((CH,), jnp.int32),
        pltpu.VMEM((CH,), jnp.float32),
        pltpu.VMEM((CH,), jnp.float32),
        pltpu.VMEM((CH,), jnp.float32),
        pltpu.VMEM((CH,), jnp.float32),
        pltpu.VMEM((CH, D), jnp.float32),
        pltpu.VMEM((CH, D), jnp.float32),
        pltpu.VMEM((CH, D), jnp.float32),
        pltpu.VMEM((CH, D), jnp.float32),
        pltpu.SemaphoreType.DMA,
        pltpu.SemaphoreType.DMA,
        pltpu.SemaphoreType.DMA,
        pltpu.SemaphoreType.DMA,
        pltpu.SemaphoreType.DMA,
        pltpu.SemaphoreType.DMA,
        pltpu.SemaphoreType.DMA,
        pltpu.SemaphoreType.DMA,
        pltpu.SemaphoreType.DMA,
        pltpu.VMEM_SHARED((PAD, D), jnp.float32),
    ],
)

UPW = BATCH // (NC * NS)


def _gusers_body(u_hbm, x0, x1, x2, x3, out_hbm, uidx_v, a_v, b_v):
    c = lax.axis_index("c")
    s = lax.axis_index("s")
    w = s * NC + c
    base = w * UPW
    pltpu.sync_copy(u_hbm.at[pl.ds(base, UPW)], uidx_v)
    pltpu.sync_copy(x0.at[uidx_v], a_v)
    for t in (x1, x2, x3):
        pltpu.sync_copy(t.at[uidx_v], b_v)

        def addr(r, _):
            for q in range(D // L):
                sl = pl.ds(q * L, L)
                a_v[r, sl] = a_v[r, sl] + b_v[r, sl]
            return 0
        lax.fori_loop(0, UPW, addr, 0)
    pltpu.sync_copy(a_v, out_hbm.at[pl.ds(base, UPW)])


_gather_users = pl.kernel(
    _gusers_body,
    out_type=jax.ShapeDtypeStruct((BATCH, D), jnp.float32),
    mesh=_mesh,
    compiler_params=_sc_params,
    scratch_types=[
        pltpu.VMEM((UPW,), jnp.int32),
        pltpu.VMEM((UPW, D), jnp.float32),
        pltpu.VMEM((UPW, D), jnp.float32),
    ],
)

SUMB = 1024


def _sum_body(a, b, c, d, o):
    o[...] = a[...] + b[...] + c[...] + d[...]


def _sum_items(x0, x1, x2, x3):
    in_spec = pl.BlockSpec((SUMB, D), lambda j: (PAD // SUMB + j, 0))
    return pl.pallas_call(
        _sum_body,
        grid=(PAD // SUMB,),
        in_specs=[in_spec, in_spec, in_spec, in_spec],
        out_specs=pl.BlockSpec((SUMB, D), lambda j: (j, 0)),
        out_shape=jax.ShapeDtypeStruct((PAD, D), jnp.float32),
    )(x0, x1, x2, x3)


MB = 64


def _mm_body(u_ref, it_ref, o_ref):
    acc = lax.dot_general(u_ref[...], it_ref[:NU, :],
                          (((1,), (1,)), ((), ())),
                          preferred_element_type=jnp.float32)
    o_ref[...] = jax.nn.sigmoid(acc * 0.0625)


def _rating(u_s, items_sum):
    return pl.pallas_call(
        _mm_body,
        grid=(BATCH // MB,),
        in_specs=[pl.BlockSpec((MB, D), lambda j: (j, 0)),
                  pl.BlockSpec((PAD, D), lambda j: (0, 0))],
        out_specs=pl.BlockSpec((MB, NU), lambda j: (j, 0)),
        out_shape=jax.ShapeDtypeStruct((BATCH, NU), jnp.float32),
    )(u_s, items_sum)


@jax.jit
def kernel(users, edge_index, edge_weight, user_emb, item_emb):
    users = users.astype(jnp.int32)
    src = edge_index[0].astype(jnp.int32)
    dst = edge_index[1].astype(jnp.int32)
    w = edge_weight.astype(jnp.float32)

    x0 = jnp.zeros((TBL, D), jnp.float32)
    x0 = x0.at[:NU].set(user_emb).at[PAD:PAD + NU].set(item_emb)

    x1 = _propagate(x0, src, dst, w)
    x2 = _propagate(x1, src, dst, w)
    x3 = _propagate(x2, src, dst, w)

    i_sum = _sum_items(x0, x1, x2, x3)
    u_s = _gather_users(users, x0, x1, x2, x3)
    return _rating(u_s, i_sum)


# speedup vs baseline: 2.9073x; 1.0116x over previous
import functools

import jax
import jax.numpy as jnp
from jax import lax
from jax.experimental import pallas as pl
from jax.experimental.pallas import tpu as pltpu
from jax.experimental.pallas import tpu_sc as plsc

NU = 25000
PAD = 25600
TBL = 2 * PAD
D = 64
E = 800000
BATCH = 1024
NC, NS, L = 2, 16, 16
EPT = E // NS
CH = 80
STRIPE = PAD // NS
BLK = 2000
CPB = BLK // CH
NB = EPT // BLK

_mesh = plsc.VectorSubcoreMesh(core_axis_name="c", subcore_axis_name="s")
_sc_params = pltpu.CompilerParams(needs_layout_passes=False,
                                  use_tc_tiling_on_sc=False)


def _prop_body(x_hbm, src_hbm, dst_hbm, w_hbm, out_hbm,
               src_b, dst_b, w_b,
               sv0, sv1, sv2, sv3, iv0, iv1, iv2, iv3,
               wv0, wv1, wv2, wv3,
               r0, r1, r2, r3,
               g0, g1, g2, g3, s0, s1, s2, s3, esem, acc):
    SV = (sv0, sv1, sv2, sv3)
    IV = (iv0, iv1, iv2, iv3)
    WV = (wv0, wv1, wv2, wv3)
    RW = (r0, r1, r2, r3)
    GS = (g0, g1, g2, g3)
    SS = (s0, s1, s2, s3)
    c = lax.axis_index("c")
    s = lax.axis_index("s")
    nu = jnp.int32(NU)

    @plsc.parallel_loop(0, CH, unroll=4)
    def zrow(r):
        z = jnp.zeros((L,), jnp.float32)
        for q in range(D // L):
            r0[r, pl.ds(q * L, L)] = z

    def zcp(k, _):
        pltpu.sync_copy(r0, acc.at[pl.ds(s * STRIPE + k * CH, CH)])
        return 0
    lax.fori_loop(0, STRIPE // CH, zcp, 0)

    trash16 = nu + lax.iota(jnp.int32, L)
    for ivb in IV:
        def tini(i, _, ivb=ivb):
            ivb[pl.ds(i * L, L)] = trash16
            return 0
        lax.fori_loop(0, CH // L, tini, 0)
    plsc.subcore_barrier()
    for b in range(4):
        pltpu.async_copy(RW[b], acc.at[IV[b]], SS[b], add=True)

    base_node = c * nu

    def remap(ck, src_v, idx_v, w_v):
        @plsc.parallel_loop(0, CH // L, unroll=2)
        def body(i):
            sl_b = pl.ds(ck * CH + i * L, L)
            sl = pl.ds(i * L, L)
            sv = src_b[sl_b]
            src_v[sl] = jnp.where(sv >= nu, sv + (PAD - NU), sv)
            dv = dst_b[sl_b]
            ld = dv - base_node
            ok = (ld >= 0) & (ld < nu)
            idx_v[sl] = jnp.where(ok, ld, nu + (dv & 511))
            w_v[sl] = w_b[sl_b]

    def scale(rows, w_v):
        @plsc.parallel_loop(0, CH // 4, unroll=4)
        def body(jj):
            for u in range(4):
                j = jj * 4 + u
                wspl = plsc.load_gather(
                    w_v, [jnp.zeros((L,), jnp.int32) + j])
                for q in range(D // L):
                    sl = pl.ds(q * L, L)
                    rows[j, sl] = rows[j, sl] * wspl

    def wait_scatter(b):
        pltpu.make_async_copy(RW[b], acc.at[IV[b]], SS[b]).wait()

    def wait_gather(b):
        pltpu.make_async_copy(x_hbm.at[SV[b]], RW[b], GS[b]).wait()

    def slot(ck, t):
        b2 = (t + 2) % 4

        @pl.when(ck + 2 < CPB)
        def _():
            wait_scatter(b2)
            remap(ck + 2, SV[b2], IV[b2], WV[b2])
            pltpu.async_copy(x_hbm.at[SV[b2]], RW[b2], GS[b2])
        wait_gather(t)
        scale(RW[t], WV[t])
        pltpu.async_copy(RW[t], acc.at[IV[t]], SS[t], add=True)

    def edge_load(blk, copy):
        base = s * EPT + blk * BLK
        copy(src_hbm.at[pl.ds(base, BLK)], src_b)
        copy(dst_hbm.at[pl.ds(base, BLK)], dst_b)
        copy(w_hbm.at[pl.ds(base, BLK)], w_b)

    edge_load(0, lambda a, b: pltpu.async_copy(a, b, esem))

    def block(blk, _):
        edge_load(blk, lambda a, b: pltpu.make_async_copy(a, b, esem).wait())

        for b in (0, 1):
            wait_scatter(b)
            remap(b, SV[b], IV[b], WV[b])
            pltpu.async_copy(x_hbm.at[SV[b]], RW[b], GS[b])

        def quad(q, _):
            for t in range(4):
                slot(4 * q + t, t)
            return 0
        lax.fori_loop(0, CPB // 4, quad, 0)

        @pl.when(blk + 1 < NB)
        def _():
            edge_load(blk + 1, lambda a, b: pltpu.async_copy(a, b, esem))

        wait_gather(0)
        scale(RW[0], WV[0])
        pltpu.async_copy(RW[0], acc.at[IV[0]], SS[0], add=True)
        return 0

    lax.fori_loop(0, NB, block, 0)

    for b in range(4):
        wait_scatter(b)
    plsc.subcore_barrier()

    pltpu.sync_copy(acc.at[pl.ds(s * STRIPE, STRIPE)],
                    out_hbm.at[pl.ds(c * PAD + s * STRIPE, STRIPE)])


_propagate = pl.kernel(
    _prop_body,
    out_type=jax.ShapeDtypeStruct((TBL, D), jnp.float32),
    mesh=_mesh,
    compiler_params=_sc_params,
    scratch_types=[
        pltpu.VMEM((BLK,), jnp.int32),
        pltpu.VMEM((BLK,), jnp.int32),
        pltpu.VMEM((BLK,), jnp.float32),
        pltpu.VMEM((CH,), jnp.int32),
        pltpu.VMEM((CH,), jnp.int32),
        pltpu.VMEM((CH,), jnp.int32),
        pltpu.VMEM((CH,), jnp.int32),
        pltpu.VMEM((CH,), jnp.int32),
        pltpu.VMEM((CH,), jnp.int32),
        pltpu.VMEM((CH,), jnp.int32),
        pltpu.VMEM((CH,), jnp.int32),
        pltpu.VMEM((CH,), jnp.float32),
        pltpu.VMEM((CH,), jnp.float32),
        pltpu.VMEM((CH,), jnp.float32),
        pltpu.VMEM((CH,), jnp.float32),
        pltpu.VMEM((CH, D), jnp.float32),
        pltpu.VMEM((CH, D), jnp.float32),
        pltpu.VMEM((CH, D), jnp.float32),
        pltpu.VMEM((CH, D), jnp.float32),
        pltpu.SemaphoreType.DMA,
        pltpu.SemaphoreType.DMA,
        pltpu.SemaphoreType.DMA,
        pltpu.SemaphoreType.DMA,
        pltpu.SemaphoreType.DMA,
        pltpu.SemaphoreType.DMA,
        pltpu.SemaphoreType.DMA,
        pltpu.SemaphoreType.DMA,
        pltpu.SemaphoreType.DMA,
        pltpu.VMEM_SHARED((PAD, D), jnp.float32),
    ],
)

UPW = BATCH // (NC * NS)
ISTR = PAD // (NC * NS)
ICH = 80


def _tail_body(u_hbm, x0, x1, x2, x3, isum_hbm, us_hbm,
               uidx_v, a_v, b_v, i0, i1, i2, i3):
    c = lax.axis_index("c")
    s = lax.axis_index("s")
    w = s * NC + c
    base = w * UPW

    pltpu.sync_copy(u_hbm.at[pl.ds(base, UPW)], uidx_v)
    pltpu.sync_copy(x0.at[uidx_v], a_v)
    for t in (x1, x2, x3):
        pltpu.sync_copy(t.at[uidx_v], b_v)

        @plsc.parallel_loop(0, UPW, unroll=2)
        def addr(r):
            for q in range(D // L):
                sl = pl.ds(q * L, L)
                a_v[r, sl] = a_v[r, sl] + b_v[r, sl]
    pltpu.sync_copy(a_v, us_hbm.at[pl.ds(base, UPW)])

    ibase = w * ISTR

    def ichunk(k, _):
        rb = PAD + ibase + k * ICH
        pltpu.sync_copy(x0.at[pl.ds(rb, ICH)], i0)
        pltpu.sync_copy(x1.at[pl.ds(rb, ICH)], i1)
        pltpu.sync_copy(x2.at[pl.ds(rb, ICH)], i2)
        pltpu.sync_copy(x3.at[pl.ds(rb, ICH)], i3)

        @plsc.parallel_loop(0, ICH, unroll=2)
        def addr(r):
            for q in range(D // L):
                sl = pl.ds(q * L, L)
                i0[r, sl] = (i0[r, sl] + i1[r, sl]) + (i2[r, sl] + i3[r, sl])
        pltpu.sync_copy(i0, isum_hbm.at[pl.ds(ibase + k * ICH, ICH)])
        return 0
    lax.fori_loop(0, ISTR // ICH, ichunk, 0)


_tail_sc = pl.kernel(
    _tail_body,
    out_type=(jax.ShapeDtypeStruct((PAD, D), jnp.float32),
              jax.ShapeDtypeStruct((BATCH, D), jnp.float32)),
    mesh=_mesh,
    compiler_params=_sc_params,
    scratch_types=[
        pltpu.VMEM((UPW,), jnp.int32),
        pltpu.VMEM((UPW, D), jnp.float32),
        pltpu.VMEM((UPW, D), jnp.float32),
        pltpu.VMEM((ICH, D), jnp.float32),
        pltpu.VMEM((ICH, D), jnp.float32),
        pltpu.VMEM((ICH, D), jnp.float32),
        pltpu.VMEM((ICH, D), jnp.float32),
    ],
)


MB = 64


def _mm_body(u_ref, it_ref, o_ref):
    acc = lax.dot_general(u_ref[...], it_ref[:NU, :],
                          (((1,), (1,)), ((), ())),
                          preferred_element_type=jnp.float32)
    o_ref[...] = jax.nn.sigmoid(acc * 0.0625)


def _rating(u_s, items_sum):
    return pl.pallas_call(
        _mm_body,
        grid=(BATCH // MB,),
        in_specs=[pl.BlockSpec((MB, D), lambda j: (j, 0)),
                  pl.BlockSpec((PAD, D), lambda j: (0, 0))],
        out_specs=pl.BlockSpec((MB, NU), lambda j: (j, 0)),
        out_shape=jax.ShapeDtypeStruct((BATCH, NU), jnp.float32),
    )(u_s, items_sum)


@jax.jit
def kernel(users, edge_index, edge_weight, user_emb, item_emb):
    users = users.astype(jnp.int32)
    src = edge_index[0].astype(jnp.int32)
    dst = edge_index[1].astype(jnp.int32)
    w = edge_weight.astype(jnp.float32)

    x0 = jnp.zeros((TBL, D), jnp.float32)
    x0 = x0.at[:NU].set(user_emb).at[PAD:PAD + NU].set(item_emb)

    x1 = _propagate(x0, src, dst, w)
    x2 = _propagate(x1, src, dst, w)
    x3 = _propagate(x2, src, dst, w)

    i_sum, u_s = _tail_sc(users, x0, x1, x2, x3)
    return _rating(u_s, i_sum)


# speedup vs baseline: 2.9362x; 1.0099x over previous
import functools

import jax
import jax.numpy as jnp
from jax import lax
from jax.experimental import pallas as pl
from jax.experimental.pallas import tpu as pltpu
from jax.experimental.pallas import tpu_sc as plsc

NU = 25000
PAD = 25600
TBL = 2 * PAD
D = 64
E = 800000
BATCH = 1024
NC, NS, L = 2, 16, 16
EPT = E // NS
CH = 80
STRIPE = PAD // NS
BLK = 2000
CPB = BLK // CH
NB = EPT // BLK

_mesh = plsc.VectorSubcoreMesh(core_axis_name="c", subcore_axis_name="s")
_sc_params = pltpu.CompilerParams(needs_layout_passes=False,
                                  use_tc_tiling_on_sc=False)


def _prop_body(x_hbm, src_hbm, dst_hbm, w_hbm, out_hbm,
               src_b, dst_b, w_b,
               sv0, sv1, sv2, sv3, iv0, iv1, iv2, iv3,
               wv0, wv1, wv2, wv3,
               r0, r1, r2, r3,
               g0, g1, g2, g3, s0, s1, s2, s3, esem, acc):
    SV = (sv0, sv1, sv2, sv3)
    IV = (iv0, iv1, iv2, iv3)
    WV = (wv0, wv1, wv2, wv3)
    RW = (r0, r1, r2, r3)
    GS = (g0, g1, g2, g3)
    SS = (s0, s1, s2, s3)
    c = lax.axis_index("c")
    s = lax.axis_index("s")
    nu = jnp.int32(NU)

    @plsc.parallel_loop(0, CH, unroll=4)
    def zrow(r):
        z = jnp.zeros((L,), jnp.float32)
        for q in range(D // L):
            r0[r, pl.ds(q * L, L)] = z

    def zcp(k, _):
        pltpu.async_copy(r0, acc.at[pl.ds(s * STRIPE + k * CH, CH)], esem)
        return 0
    lax.fori_loop(0, STRIPE // CH, zcp, 0)

    def zwait(k, _):
        pltpu.make_async_copy(
            r0, acc.at[pl.ds(s * STRIPE + k * CH, CH)], esem).wait()
        return 0
    lax.fori_loop(0, STRIPE // CH, zwait, 0)

    trash16 = nu + lax.iota(jnp.int32, L)
    for ivb in IV:
        def tini(i, _, ivb=ivb):
            ivb[pl.ds(i * L, L)] = trash16
            return 0
        lax.fori_loop(0, CH // L, tini, 0)
    plsc.subcore_barrier()
    for b in range(4):
        pltpu.async_copy(RW[b], acc.at[IV[b]], SS[b], add=True)

    base_node = c * nu

    def remap(ck, src_v, idx_v, w_v):
        @plsc.parallel_loop(0, CH // L, unroll=2)
        def body(i):
            sl_b = pl.ds(ck * CH + i * L, L)
            sl = pl.ds(i * L, L)
            sv = src_b[sl_b]
            src_v[sl] = jnp.where(sv >= nu, sv + (PAD - NU), sv)
            dv = dst_b[sl_b]
            ld = dv - base_node
            ok = (ld >= 0) & (ld < nu)
            idx_v[sl] = jnp.where(ok, ld, nu + (dv & 511))
            w_v[sl] = w_b[sl_b]

    def scale(rows, w_v):
        @plsc.parallel_loop(0, CH // 4, unroll=4)
        def body(jj):
            for u in range(4):
                j = jj * 4 + u
                wspl = plsc.load_gather(
                    w_v, [jnp.zeros((L,), jnp.int32) + j])
                for q in range(D // L):
                    sl = pl.ds(q * L, L)
                    rows[j, sl] = rows[j, sl] * wspl

    def wait_scatter(b):
        pltpu.make_async_copy(RW[b], acc.at[IV[b]], SS[b]).wait()

    def wait_gather(b):
        pltpu.make_async_copy(x_hbm.at[SV[b]], RW[b], GS[b]).wait()

    def slot(ck, t):
        b2 = (t + 2) % 4

        @pl.when(ck + 2 < CPB)
        def _():
            wait_scatter(b2)
            remap(ck + 2, SV[b2], IV[b2], WV[b2])
            pltpu.async_copy(x_hbm.at[SV[b2]], RW[b2], GS[b2])
        wait_gather(t)
        scale(RW[t], WV[t])
        pltpu.async_copy(RW[t], acc.at[IV[t]], SS[t], add=True)

    def edge_load(blk, copy):
        base = s * EPT + blk * BLK
        copy(src_hbm.at[pl.ds(base, BLK)], src_b)
        copy(dst_hbm.at[pl.ds(base, BLK)], dst_b)
        copy(w_hbm.at[pl.ds(base, BLK)], w_b)

    edge_load(0, lambda a, b: pltpu.async_copy(a, b, esem))

    def block(blk, _):
        edge_load(blk, lambda a, b: pltpu.make_async_copy(a, b, esem).wait())

        for b in (0, 1):
            wait_scatter(b)
            remap(b, SV[b], IV[b], WV[b])
            pltpu.async_copy(x_hbm.at[SV[b]], RW[b], GS[b])

        def quad(q, _):
            for t in range(4):
                slot(4 * q + t, t)
            return 0
        lax.fori_loop(0, CPB // 4, quad, 0)

        @pl.when(blk + 1 < NB)
        def _():
            edge_load(blk + 1, lambda a, b: pltpu.async_copy(a, b, esem))

        wait_gather(0)
        scale(RW[0], WV[0])
        pltpu.async_copy(RW[0], acc.at[IV[0]], SS[0], add=True)
        return 0

    lax.fori_loop(0, NB, block, 0)

    for b in range(4):
        wait_scatter(b)
    plsc.subcore_barrier()

    pltpu.sync_copy(acc.at[pl.ds(s * STRIPE, STRIPE)],
                    out_hbm.at[pl.ds(c * PAD + s * STRIPE, STRIPE)])


_propagate = pl.kernel(
    _prop_body,
    out_type=jax.ShapeDtypeStruct((TBL, D), jnp.float32),
    mesh=_mesh,
    compiler_params=_sc_params,
    scratch_types=[
        pltpu.VMEM((BLK,), jnp.int32),
        pltpu.VMEM((BLK,), jnp.int32),
        pltpu.VMEM((BLK,), jnp.float32),
        pltpu.VMEM((CH,), jnp.int32),
        pltpu.VMEM((CH,), jnp.int32),
        pltpu.VMEM((CH,), jnp.int32),
        pltpu.VMEM((CH,), jnp.int32),
        pltpu.VMEM((CH,), jnp.int32),
        pltpu.VMEM((CH,), jnp.int32),
        pltpu.VMEM((CH,), jnp.int32),
        pltpu.VMEM((CH,), jnp.int32),
        pltpu.VMEM((CH,), jnp.float32),
        pltpu.VMEM((CH,), jnp.float32),
        pltpu.VMEM((CH,), jnp.float32),
        pltpu.VMEM((CH,), jnp.float32),
        pltpu.VMEM((CH, D), jnp.float32),
        pltpu.VMEM((CH, D), jnp.float32),
        pltpu.VMEM((CH, D), jnp.float32),
        pltpu.VMEM((CH, D), jnp.float32),
        pltpu.SemaphoreType.DMA,
        pltpu.SemaphoreType.DMA,
        pltpu.SemaphoreType.DMA,
        pltpu.SemaphoreType.DMA,
        pltpu.SemaphoreType.DMA,
        pltpu.SemaphoreType.DMA,
        pltpu.SemaphoreType.DMA,
        pltpu.SemaphoreType.DMA,
        pltpu.SemaphoreType.DMA,
        pltpu.VMEM_SHARED((PAD, D), jnp.float32),
    ],
)

UPW = BATCH // (NC * NS)
ISTR = PAD // (NC * NS)
ICH = 80


def _tail_body(u_hbm, x0, x1, x2, x3, isum_hbm, us_hbm,
               uidx_v, a_v, b_v, i0, i1, i2, i3):
    c = lax.axis_index("c")
    s = lax.axis_index("s")
    w = s * NC + c
    base = w * UPW

    pltpu.sync_copy(u_hbm.at[pl.ds(base, UPW)], uidx_v)
    pltpu.sync_copy(x0.at[uidx_v], a_v)
    for t in (x1, x2, x3):
        pltpu.sync_copy(t.at[uidx_v], b_v)

        @plsc.parallel_loop(0, UPW, unroll=2)
        def addr(r):
            for q in range(D // L):
                sl = pl.ds(q * L, L)
                a_v[r, sl] = a_v[r, sl] + b_v[r, sl]
    pltpu.sync_copy(a_v, us_hbm.at[pl.ds(base, UPW)])

    ibase = w * ISTR

    def ichunk(k, _):
        rb = PAD + ibase + k * ICH
        pltpu.sync_copy(x0.at[pl.ds(rb, ICH)], i0)
        pltpu.sync_copy(x1.at[pl.ds(rb, ICH)], i1)
        pltpu.sync_copy(x2.at[pl.ds(rb, ICH)], i2)
        pltpu.sync_copy(x3.at[pl.ds(rb, ICH)], i3)

        @plsc.parallel_loop(0, ICH, unroll=2)
        def addr(r):
            for q in range(D // L):
                sl = pl.ds(q * L, L)
                i0[r, sl] = (i0[r, sl] + i1[r, sl]) + (i2[r, sl] + i3[r, sl])
        pltpu.sync_copy(i0, isum_hbm.at[pl.ds(ibase + k * ICH, ICH)])
        return 0
    lax.fori_loop(0, ISTR // ICH, ichunk, 0)


_tail_sc = pl.kernel(
    _tail_body,
    out_type=(jax.ShapeDtypeStruct((PAD, D), jnp.float32),
              jax.ShapeDtypeStruct((BATCH, D), jnp.float32)),
    mesh=_mesh,
    compiler_params=_sc_params,
    scratch_types=[
        pltpu.VMEM((UPW,), jnp.int32),
        pltpu.VMEM((UPW, D), jnp.float32),
        pltpu.VMEM((UPW, D), jnp.float32),
        pltpu.VMEM((ICH, D), jnp.float32),
        pltpu.VMEM((ICH, D), jnp.float32),
        pltpu.VMEM((ICH, D), jnp.float32),
        pltpu.VMEM((ICH, D), jnp.float32),
    ],
)


MB = 128


def _mm_body(u_ref, it_ref, o_ref):
    acc = lax.dot_general(u_ref[...], it_ref[:NU, :],
                          (((1,), (1,)), ((), ())),
                          preferred_element_type=jnp.float32)
    o_ref[...] = jax.nn.sigmoid(acc * 0.0625)


def _rating(u_s, items_sum):
    return pl.pallas_call(
        _mm_body,
        grid=(BATCH // MB,),
        in_specs=[pl.BlockSpec((MB, D), lambda j: (j, 0)),
                  pl.BlockSpec((PAD, D), lambda j: (0, 0))],
        out_specs=pl.BlockSpec((MB, NU), lambda j: (j, 0)),
        out_shape=jax.ShapeDtypeStruct((BATCH, NU), jnp.float32),
    )(u_s, items_sum)


@jax.jit
def kernel(users, edge_index, edge_weight, user_emb, item_emb):
    users = users.astype(jnp.int32)
    src = edge_index[0].astype(jnp.int32)
    dst = edge_index[1].astype(jnp.int32)
    w = edge_weight.astype(jnp.float32)

    x0 = jnp.zeros((TBL, D), jnp.float32)
    x0 = x0.at[:NU].set(user_emb).at[PAD:PAD + NU].set(item_emb)

    x1 = _propagate(x0, src, dst, w)
    x2 = _propagate(x1, src, dst, w)
    x3 = _propagate(x2, src, dst, w)

    i_sum, u_s = _tail_sc(users, x0, x1, x2, x3)
    return _rating(u_s, i_sum)
